# baseline (speedup 1.0000x reference)
import functools

import jax
import jax.numpy as jnp
from jax.experimental import pallas as pl
from jax.experimental.pallas import tpu as pltpu
from jax.experimental.pallas import tpu_sc as plsc

NUM_CLASSES = 117
NUM_OBJ = 80
N_DET = 1000
N_HUM = 16
POOL = 4
NPTS = POOL * POOL
C = 64
FH = FW = 64
THRESH = 0.2
P = N_HUM * N_DET
NIDX = P * NPTS
REP = 128
GATHER_WIN = 256
CHUNKS = 4


def _idx_body(boxes_ref, out_ref):
    h = pl.program_id(0)
    b = boxes_ref[...]
    rowi = jax.lax.broadcasted_iota(jnp.int32, (N_DET, 1), 0)
    hm = rowi == h
    bh = jnp.sum(jnp.where(hm, b, 0.0), axis=0, keepdims=True)
    x1h, y1h, x2h, y2h = (bh[:, 0:1], bh[:, 1:2], bh[:, 2:3], bh[:, 3:4])
    x1o, y1o, x2o, y2o = (b[:, 0:1], b[:, 1:2], b[:, 2:3], b[:, 3:4])
    ux1 = jnp.minimum(x1h, x1o)
    uy1 = jnp.minimum(y1h, y1o)
    ux2 = jnp.maximum(x2h, x2o)
    uy2 = jnp.maximum(y2h, y2o)
    chunks = []
    for g8 in range(NPTS // 2):
        i_pt, j0 = g8 // 2, 2 * (g8 % 2)
        fy = (i_pt + 0.5) / POOL
        fx0 = (j0 + 0.5) / POOL
        fx1 = (j0 + 1.5) / POOL
        yi = jnp.clip(jnp.round(uy1 + (uy2 - uy1) * fy), 0.0, FH - 1)
        xi0 = jnp.clip(jnp.round(ux1 + (ux2 - ux1) * fx0), 0.0, FW - 1)
        xi1 = jnp.clip(jnp.round(ux1 + (ux2 - ux1) * fx1), 0.0, FW - 1)
        chunks.append(jnp.concatenate([yi * FW + xi0, yi * FW + xi1], axis=1))
    out_ref[...] = jnp.concatenate(chunks, axis=0).astype(jnp.int32)


def _pair_indices(boxes):
    return pl.pallas_call(
        _idx_body,
        grid=(N_HUM,),
        in_specs=[pl.BlockSpec((N_DET, 4), lambda h: (0, 0))],
        out_specs=pl.BlockSpec((8 * N_DET, 2), lambda h: (h, 0)),
        out_shape=jax.ShapeDtypeStruct((P * 8, 2), jnp.int32),
    )(boxes)


def _sc_gather(featT, idx_flat, n):
    mesh = plsc.VectorSubcoreMesh(core_axis_name="c", subcore_axis_name="s")

    @functools.partial(
        pl.kernel,
        out_type=jax.ShapeDtypeStruct((n, C), jnp.float32),
        mesh=mesh,
        compiler_params=pltpu.CompilerParams(use_tc_tiling_on_sc=False),
    )
    def gk(x_hbm, i_hbm, o_hbm):
        def body(i_vmem, o_vmem):
            pltpu.sync_copy(x_hbm.at[i_vmem.at[0]], o_vmem)

        nchunks = n // GATHER_WIN
        pltpu.emit_pipeline(
            body,
            grid=(2, nchunks // 2),
            in_specs=[
                pl.BlockSpec(
                    (1, GATHER_WIN),
                    lambda i, j: (0, i * (nchunks // 2) + j),
                )
            ],
            out_specs=[
                pl.BlockSpec(
                    (GATHER_WIN, C),
                    lambda i, j: (i * (nchunks // 2) + j, 0),
                )
            ],
            core_axis_name=("c", "s"),
            dimension_semantics=(pltpu.PARALLEL, pltpu.PARALLEL),
        )(i_hbm, o_hbm)

    return gk(featT, idx_flat)


def _head_body(h_base, x_ref, sc_ref, lab_ref, o2t_ref,
               w1_ref, b1_ref, w2_ref, b2_ref, w3_ref, b3_ref, out_ref):
    h = pl.program_id(0) + h_base
    rowi = jax.lax.broadcasted_iota(jnp.int32, (N_DET, 1), 0)
    hm = rowi == h
    s = sc_ref[...]
    se = jnp.where(s >= THRESH, s, 0.0)
    sh = jnp.sum(jnp.where(hm, se, 0.0))
    ds = sh * se * jnp.where(hm, 0.0, 1.0)
    lab = lab_ref[...]
    l_iota = jax.lax.broadcasted_iota(jnp.int32, (N_DET, NUM_OBJ), 1).astype(
        jnp.float32)
    ohl = jnp.where(lab == l_iota, 1.0, 0.0)
    tgt = jnp.sum(ohl * o2t_ref[...], axis=1, keepdims=True)

    x8 = x_ref[...]
    x = jnp.concatenate(
        [x8[g8 * N_DET:(g8 + 1) * N_DET, :] for g8 in range(NPTS // 2)],
        axis=1,
    )
    h1 = jax.nn.relu(
        jnp.dot(x, w1_ref[...], preferred_element_type=jnp.float32)
        + b1_ref[...]
    )
    h2 = jax.nn.relu(
        jnp.dot(h1, w2_ref[...], preferred_element_type=jnp.float32)
        + b2_ref[...]
    )
    logits = (
        jnp.dot(h2, w3_ref[...], preferred_element_type=jnp.float32)
        + b3_ref[...]
    )
    k_iota = jax.lax.broadcasted_iota(jnp.int32, (N_DET, NUM_CLASSES), 1
                                      ).astype(jnp.float32)
    onehot = jnp.where(tgt == k_iota, 1.0, 0.0)
    out_ref[...] = ds * onehot * jax.nn.sigmoid(logits)


def _head(h_base, n_h, x2d, scores_c, labels_f, o2t_f, W1p, b1r, W2, b2r,
          W3, b3r):
    full = lambda shape: pl.BlockSpec(shape, lambda h: (0, 0))
    return pl.pallas_call(
        functools.partial(_head_body, h_base),
        grid=(n_h,),
        in_specs=[
            pl.BlockSpec((8 * N_DET, 2 * C), lambda h: (h, 0)),
            full((N_DET, 1)),
            full((N_DET, 1)),
            full((1, NUM_OBJ)),
            full((C * NPTS, REP)),
            full((1, REP)),
            full((REP, REP)),
            full((1, REP)),
            full((REP, NUM_CLASSES)),
            full((1, NUM_CLASSES)),
        ],
        out_specs=pl.BlockSpec((N_DET, NUM_CLASSES), lambda h: (h, 0)),
        out_shape=jax.ShapeDtypeStruct((n_h * N_DET, NUM_CLASSES),
                                       jnp.float32),
    )(x2d, scores_c, labels_f, o2t_f, W1p, b1r, W2, b2r, W3, b3r)


def kernel(features, boxes, scores, labels, W1, b1, W2, b2, W3, b3, obj2target):
    featT = features.transpose(1, 2, 0).reshape(FH * FW, C)
    W1p = (W1.reshape(C, NPTS // 2, 2, REP).transpose(1, 2, 0, 3)
           .reshape(C * NPTS, REP))
    idx = _pair_indices(boxes)
    scores_c = scores.reshape(N_DET, 1)
    labels_f = labels.astype(jnp.float32).reshape(N_DET, 1)
    o2t_f = obj2target.astype(jnp.float32).reshape(1, NUM_OBJ)
    b1r, b2r, b3r = b1.reshape(1, REP), b2.reshape(1, REP), b3.reshape(1, NUM_CLASSES)
    h_per = N_HUM // CHUNKS
    rows_per = h_per * N_DET * 8
    n_per = NIDX // CHUNKS
    outs = []
    for k in range(CHUNKS):
        idx_k = idx[k * rows_per:(k + 1) * rows_per].reshape(1, n_per)
        pooled = _sc_gather(featT, idx_k, n_per)
        x2d = pooled.reshape(n_per // 2, 2 * C)
        outs.append(_head(k * h_per, h_per, x2d, scores_c, labels_f, o2t_f,
                          W1p, b1r, W2, b2r, W3, b3r))
    return jnp.concatenate(outs, axis=0)


# speedup vs baseline: 1.0588x; 1.0588x over previous
import functools

import jax
import jax.numpy as jnp
from jax.experimental import pallas as pl
from jax.experimental.pallas import tpu as pltpu
from jax.experimental.pallas import tpu_sc as plsc

NUM_CLASSES = 117
NUM_OBJ = 80
N_DET = 1000
N_HUM = 16
POOL = 4
NPTS = POOL * POOL
C = 64
FH = FW = 64
THRESH = 0.2
P = N_HUM * N_DET
NIDX = P * NPTS
REP = 128
GATHER_WIN = 256
CHUNKS = 2


def _idx_body(boxes_ref, out_ref):
    h = pl.program_id(0)
    b = boxes_ref[...]
    rowi = jax.lax.broadcasted_iota(jnp.int32, (N_DET, 1), 0)
    hm = rowi == h
    bh = jnp.sum(jnp.where(hm, b, 0.0), axis=0, keepdims=True)
    x1h, y1h, x2h, y2h = (bh[:, 0:1], bh[:, 1:2], bh[:, 2:3], bh[:, 3:4])
    x1o, y1o, x2o, y2o = (b[:, 0:1], b[:, 1:2], b[:, 2:3], b[:, 3:4])
    ux1 = jnp.minimum(x1h, x1o)
    uy1 = jnp.minimum(y1h, y1o)
    ux2 = jnp.maximum(x2h, x2o)
    uy2 = jnp.maximum(y2h, y2o)
    chunks = []
    for g8 in range(NPTS // 2):
        i_pt, j0 = g8 // 2, 2 * (g8 % 2)
        fy = (i_pt + 0.5) / POOL
        fx0 = (j0 + 0.5) / POOL
        fx1 = (j0 + 1.5) / POOL
        yi = jnp.clip(jnp.round(uy1 + (uy2 - uy1) * fy), 0.0, FH - 1)
        xi0 = jnp.clip(jnp.round(ux1 + (ux2 - ux1) * fx0), 0.0, FW - 1)
        xi1 = jnp.clip(jnp.round(ux1 + (ux2 - ux1) * fx1), 0.0, FW - 1)
        chunks.append(jnp.concatenate([yi * FW + xi0, yi * FW + xi1], axis=1))
    out_ref[...] = jnp.concatenate(chunks, axis=0).astype(jnp.int32)


def _pair_indices(boxes):
    return pl.pallas_call(
        _idx_body,
        grid=(N_HUM,),
        in_specs=[pl.BlockSpec((N_DET, 4), lambda h: (0, 0))],
        out_specs=pl.BlockSpec((8 * N_DET, 2), lambda h: (h, 0)),
        out_shape=jax.ShapeDtypeStruct((P * 8, 2), jnp.int32),
    )(boxes)


def _sc_gather(featT, idx_flat, n):
    mesh = plsc.VectorSubcoreMesh(core_axis_name="c", subcore_axis_name="s")

    @functools.partial(
        pl.kernel,
        out_type=jax.ShapeDtypeStruct((n, C), jnp.float32),
        mesh=mesh,
        compiler_params=pltpu.CompilerParams(use_tc_tiling_on_sc=False),
    )
    def gk(x_hbm, i_hbm, o_hbm):
        def body(i_vmem, o_vmem):
            pltpu.sync_copy(x_hbm.at[i_vmem.at[0]], o_vmem)

        nchunks = n // GATHER_WIN
        pltpu.emit_pipeline(
            body,
            grid=(2, nchunks // 2),
            in_specs=[
                pl.BlockSpec(
                    (1, GATHER_WIN),
                    lambda i, j: (0, i * (nchunks // 2) + j),
                )
            ],
            out_specs=[
                pl.BlockSpec(
                    (GATHER_WIN, C),
                    lambda i, j: (i * (nchunks // 2) + j, 0),
                )
            ],
            core_axis_name=("c", "s"),
            dimension_semantics=(pltpu.PARALLEL, pltpu.PARALLEL),
        )(i_hbm, o_hbm)

    return gk(featT, idx_flat)


def _head_body(h_base, x_ref, sc_ref, lab_ref, o2t_ref,
               w1_ref, b1_ref, w2_ref, b2_ref, w3_ref, b3_ref, out_ref):
    h = pl.program_id(0) + h_base
    rowi = jax.lax.broadcasted_iota(jnp.int32, (N_DET, 1), 0)
    hm = rowi == h
    s = sc_ref[...]
    se = jnp.where(s >= THRESH, s, 0.0)
    sh = jnp.sum(jnp.where(hm, se, 0.0))
    ds = sh * se * jnp.where(hm, 0.0, 1.0)
    lab = lab_ref[...]
    l_iota = jax.lax.broadcasted_iota(jnp.int32, (N_DET, NUM_OBJ), 1).astype(
        jnp.float32)
    ohl = jnp.where(lab == l_iota, 1.0, 0.0)
    tgt = jnp.sum(ohl * o2t_ref[...], axis=1, keepdims=True)

    x8 = x_ref[...]
    x = jnp.concatenate(
        [x8[g8 * N_DET:(g8 + 1) * N_DET, :] for g8 in range(NPTS // 2)],
        axis=1,
    )
    h1 = jax.nn.relu(
        jnp.dot(x, w1_ref[...], preferred_element_type=jnp.float32)
        + b1_ref[...]
    )
    h2 = jax.nn.relu(
        jnp.dot(h1, w2_ref[...], preferred_element_type=jnp.float32)
        + b2_ref[...]
    )
    logits = (
        jnp.dot(h2, w3_ref[...], preferred_element_type=jnp.float32)
        + b3_ref[...]
    )
    k_iota = jax.lax.broadcasted_iota(jnp.int32, (N_DET, NUM_CLASSES), 1
                                      ).astype(jnp.float32)
    onehot = jnp.where(tgt == k_iota, 1.0, 0.0)
    out_ref[...] = ds * onehot * jax.nn.sigmoid(logits)


def _head(h_base, n_h, x2d, scores_c, labels_f, o2t_f, W1p, b1r, W2, b2r,
          W3, b3r):
    full = lambda shape: pl.BlockSpec(shape, lambda h: (0, 0))
    return pl.pallas_call(
        functools.partial(_head_body, h_base),
        grid=(n_h,),
        in_specs=[
            pl.BlockSpec((8 * N_DET, 2 * C), lambda h: (h, 0)),
            full((N_DET, 1)),
            full((N_DET, 1)),
            full((1, NUM_OBJ)),
            full((C * NPTS, REP)),
            full((1, REP)),
            full((REP, REP)),
            full((1, REP)),
            full((REP, NUM_CLASSES)),
            full((1, NUM_CLASSES)),
        ],
        out_specs=pl.BlockSpec((N_DET, NUM_CLASSES), lambda h: (h, 0)),
        out_shape=jax.ShapeDtypeStruct((n_h * N_DET, NUM_CLASSES),
                                       jnp.float32),
    )(x2d, scores_c, labels_f, o2t_f, W1p, b1r, W2, b2r, W3, b3r)


def kernel(features, boxes, scores, labels, W1, b1, W2, b2, W3, b3, obj2target):
    featT = features.transpose(1, 2, 0).reshape(FH * FW, C)
    W1p = (W1.reshape(C, NPTS // 2, 2, REP).transpose(1, 2, 0, 3)
           .reshape(C * NPTS, REP))
    idx = _pair_indices(boxes)
    scores_c = scores.reshape(N_DET, 1)
    labels_f = labels.astype(jnp.float32).reshape(N_DET, 1)
    o2t_f = obj2target.astype(jnp.float32).reshape(1, NUM_OBJ)
    b1r, b2r, b3r = b1.reshape(1, REP), b2.reshape(1, REP), b3.reshape(1, NUM_CLASSES)
    h_per = N_HUM // CHUNKS
    rows_per = h_per * N_DET * 8
    n_per = NIDX // CHUNKS
    outs = []
    for k in range(CHUNKS):
        idx_k = idx[k * rows_per:(k + 1) * rows_per].reshape(1, n_per)
        pooled = _sc_gather(featT, idx_k, n_per)
        x2d = pooled.reshape(n_per // 2, 2 * C)
        outs.append(_head(k * h_per, h_per, x2d, scores_c, labels_f, o2t_f,
                          W1p, b1r, W2, b2r, W3, b3r))
    return jnp.concatenate(outs, axis=0)


# speedup vs baseline: 1.1833x; 1.1176x over previous
import functools

import jax
import jax.numpy as jnp
from jax.experimental import pallas as pl
from jax.experimental.pallas import tpu as pltpu
from jax.experimental.pallas import tpu_sc as plsc

NUM_CLASSES = 117
NUM_OBJ = 80
N_DET = 1000
N_HUM = 16
POOL = 4
NPTS = POOL * POOL
C = 64
FH = FW = 64
THRESH = 0.2
P = N_HUM * N_DET
NIDX = P * NPTS
REP = 128
GATHER_WIN = 256
CHUNKS = 1
SC_WORKERS = 32
CW = 400


def _idx_body(boxes_ref, out_ref):
    h = pl.program_id(0)
    b = boxes_ref[...]
    rowi = jax.lax.broadcasted_iota(jnp.int32, (N_DET, 1), 0)
    hm = rowi == h
    bh = jnp.sum(jnp.where(hm, b, 0.0), axis=0, keepdims=True)
    x1h, y1h, x2h, y2h = (bh[:, 0:1], bh[:, 1:2], bh[:, 2:3], bh[:, 3:4])
    x1o, y1o, x2o, y2o = (b[:, 0:1], b[:, 1:2], b[:, 2:3], b[:, 3:4])
    ux1 = jnp.minimum(x1h, x1o)
    uy1 = jnp.minimum(y1h, y1o)
    ux2 = jnp.maximum(x2h, x2o)
    uy2 = jnp.maximum(y2h, y2o)
    chunks = []
    for g8 in range(NPTS // 2):
        i_pt, j0 = g8 // 2, 2 * (g8 % 2)
        fy = (i_pt + 0.5) / POOL
        fx0 = (j0 + 0.5) / POOL
        fx1 = (j0 + 1.5) / POOL
        yi = jnp.clip(jnp.round(uy1 + (uy2 - uy1) * fy), 0.0, FH - 1)
        xi0 = jnp.clip(jnp.round(ux1 + (ux2 - ux1) * fx0), 0.0, FW - 1)
        xi1 = jnp.clip(jnp.round(ux1 + (ux2 - ux1) * fx1), 0.0, FW - 1)
        chunks.append(jnp.concatenate([yi * FW + xi0, yi * FW + xi1], axis=1))
    out_ref[...] = jnp.concatenate(chunks, axis=0).astype(jnp.int32)


def _pair_indices(boxes):
    return pl.pallas_call(
        _idx_body,
        grid=(N_HUM,),
        in_specs=[pl.BlockSpec((N_DET, 4), lambda h: (0, 0))],
        out_specs=pl.BlockSpec((8 * N_DET, 2), lambda h: (h, 0)),
        out_shape=jax.ShapeDtypeStruct((P * 8, 2), jnp.int32),
    )(boxes)


def _sc_gather(featT, idx_flat, n):
    mesh = plsc.VectorSubcoreMesh(core_axis_name="c", subcore_axis_name="s")

    per_w = n // SC_WORKERS
    n_ch = per_w // CW

    @functools.partial(
        pl.kernel,
        out_type=jax.ShapeDtypeStruct((n, C), jnp.float32),
        mesh=mesh,
        compiler_params=pltpu.CompilerParams(use_tc_tiling_on_sc=False),
        scratch_types=[
            pltpu.VMEM((1, per_w), jnp.int32),
            pltpu.VMEM((CW, C), jnp.float32),
            pltpu.VMEM((CW, C), jnp.float32),
            pltpu.SemaphoreType.DMA,
            pltpu.SemaphoreType.DMA,
            pltpu.SemaphoreType.DMA,
            pltpu.SemaphoreType.DMA,
        ],
    )
    def gk2(x_hbm, i_hbm, o_hbm, idx_v, buf0, buf1, gs0, gs1, os0, os1):
        cid = jax.lax.axis_index("c")
        sid = jax.lax.axis_index("s")
        wid = sid * 2 + cid
        base = wid * per_w
        pltpu.sync_copy(i_hbm.at[0, pl.ds(base, per_w)], idx_v.at[0])
        bufs, gsems, osems = (buf0, buf1), (gs0, gs1), (os0, os1)
        gathers, outs = [None] * n_ch, [None] * n_ch
        for k in range(n_ch + 2):
            if k < n_ch:
                if k >= 2:
                    outs[k - 2].wait()
                gathers[k] = pltpu.make_async_copy(
                    x_hbm.at[idx_v.at[0, pl.ds(k * CW, CW)]],
                    bufs[k % 2],
                    gsems[k % 2],
                )
                gathers[k].start()
            if 1 <= k <= n_ch:
                j = k - 1
                gathers[j].wait()
                outs[j] = pltpu.make_async_copy(
                    bufs[j % 2],
                    o_hbm.at[pl.ds(base + j * CW, CW)],
                    osems[j % 2],
                )
                outs[j].start()
        outs[n_ch - 2].wait()
        outs[n_ch - 1].wait()

    return gk2(featT, idx_flat)


def _head_body(h_base, x_ref, sc_ref, lab_ref, o2t_ref,
               w1_ref, b1_ref, w2_ref, b2_ref, w3_ref, b3_ref, out_ref):
    h = pl.program_id(0) + h_base
    rowi = jax.lax.broadcasted_iota(jnp.int32, (N_DET, 1), 0)
    hm = rowi == h
    s = sc_ref[...]
    se = jnp.where(s >= THRESH, s, 0.0)
    sh = jnp.sum(jnp.where(hm, se, 0.0))
    ds = sh * se * jnp.where(hm, 0.0, 1.0)
    lab = lab_ref[...]
    l_iota = jax.lax.broadcasted_iota(jnp.int32, (N_DET, NUM_OBJ), 1).astype(
        jnp.float32)
    ohl = jnp.where(lab == l_iota, 1.0, 0.0)
    tgt = jnp.sum(ohl * o2t_ref[...], axis=1, keepdims=True)

    x8 = x_ref[...]
    x = jnp.concatenate(
        [x8[g8 * N_DET:(g8 + 1) * N_DET, :] for g8 in range(NPTS // 2)],
        axis=1,
    )
    h1 = jax.nn.relu(
        jnp.dot(x, w1_ref[...], preferred_element_type=jnp.float32)
        + b1_ref[...]
    )
    h2 = jax.nn.relu(
        jnp.dot(h1, w2_ref[...], preferred_element_type=jnp.float32)
        + b2_ref[...]
    )
    logits = (
        jnp.dot(h2, w3_ref[...], preferred_element_type=jnp.float32)
        + b3_ref[...]
    )
    k_iota = jax.lax.broadcasted_iota(jnp.int32, (N_DET, NUM_CLASSES), 1
                                      ).astype(jnp.float32)
    onehot = jnp.where(tgt == k_iota, 1.0, 0.0)
    out_ref[...] = ds * onehot * jax.nn.sigmoid(logits)


def _head(h_base, n_h, x2d, scores_c, labels_f, o2t_f, W1p, b1r, W2, b2r,
          W3, b3r):
    full = lambda shape: pl.BlockSpec(shape, lambda h: (0, 0))
    return pl.pallas_call(
        functools.partial(_head_body, h_base),
        grid=(n_h,),
        in_specs=[
            pl.BlockSpec((8 * N_DET, 2 * C), lambda h: (h, 0)),
            full((N_DET, 1)),
            full((N_DET, 1)),
            full((1, NUM_OBJ)),
            full((C * NPTS, REP)),
            full((1, REP)),
            full((REP, REP)),
            full((1, REP)),
            full((REP, NUM_CLASSES)),
            full((1, NUM_CLASSES)),
        ],
        out_specs=pl.BlockSpec((N_DET, NUM_CLASSES), lambda h: (h, 0)),
        out_shape=jax.ShapeDtypeStruct((n_h * N_DET, NUM_CLASSES),
                                       jnp.float32),
    )(x2d, scores_c, labels_f, o2t_f, W1p, b1r, W2, b2r, W3, b3r)


def kernel(features, boxes, scores, labels, W1, b1, W2, b2, W3, b3, obj2target):
    featT = features.transpose(1, 2, 0).reshape(FH * FW, C)
    W1p = (W1.reshape(C, NPTS // 2, 2, REP).transpose(1, 2, 0, 3)
           .reshape(C * NPTS, REP))
    idx = _pair_indices(boxes)
    scores_c = scores.reshape(N_DET, 1)
    labels_f = labels.astype(jnp.float32).reshape(N_DET, 1)
    o2t_f = obj2target.astype(jnp.float32).reshape(1, NUM_OBJ)
    b1r, b2r, b3r = b1.reshape(1, REP), b2.reshape(1, REP), b3.reshape(1, NUM_CLASSES)
    h_per = N_HUM // CHUNKS
    rows_per = h_per * N_DET * 8
    n_per = NIDX // CHUNKS
    outs = []
    for k in range(CHUNKS):
        idx_k = idx[k * rows_per:(k + 1) * rows_per].reshape(1, n_per)
        pooled = _sc_gather(featT, idx_k, n_per)
        x2d = pooled.reshape(n_per // 2, 2 * C)
        outs.append(_head(k * h_per, h_per, x2d, scores_c, labels_f, o2t_f,
                          W1p, b1r, W2, b2r, W3, b3r))
    return jnp.concatenate(outs, axis=0)


# speedup vs baseline: 1.2105x; 1.0230x over previous
import functools

import jax
import jax.numpy as jnp
from jax.experimental import pallas as pl
from jax.experimental.pallas import tpu as pltpu
from jax.experimental.pallas import tpu_sc as plsc

NUM_CLASSES = 117
NUM_OBJ = 80
N_DET = 1000
N_HUM = 16
POOL = 4
NPTS = POOL * POOL
C = 64
FH = FW = 64
THRESH = 0.2
P = N_HUM * N_DET
NIDX = P * NPTS
REP = 128
GATHER_WIN = 256
CHUNKS = 1
SC_WORKERS = 32
CW = 400


def _idx_body(bh_ref, boxes_ref, out_ref):
    b = boxes_ref[...]
    bh = bh_ref[0]
    x1o, y1o, x2o, y2o = (b[:, 0:1], b[:, 1:2], b[:, 2:3], b[:, 3:4])
    ux1 = jnp.minimum(bh[:, 0:1], x1o)
    uy1 = jnp.minimum(bh[:, 1:2], y1o)
    ux2 = jnp.maximum(bh[:, 2:3], x2o)
    uy2 = jnp.maximum(bh[:, 3:4], y2o)
    chunks = []
    for g8 in range(NPTS // 2):
        i_pt, j0 = g8 // 2, 2 * (g8 % 2)
        fy = (i_pt + 0.5) / POOL
        fx0 = (j0 + 0.5) / POOL
        fx1 = (j0 + 1.5) / POOL
        yi = jnp.clip(jnp.round(uy1 + (uy2 - uy1) * fy), 0.0, FH - 1)
        xi0 = jnp.clip(jnp.round(ux1 + (ux2 - ux1) * fx0), 0.0, FW - 1)
        xi1 = jnp.clip(jnp.round(ux1 + (ux2 - ux1) * fx1), 0.0, FW - 1)
        chunks.append(jnp.concatenate([yi * FW + xi0, yi * FW + xi1], axis=1))
    out_ref[...] = jnp.concatenate(chunks, axis=0).astype(jnp.int32)


def _pair_indices(boxes):
    return pl.pallas_call(
        _idx_body,
        grid=(N_HUM,),
        in_specs=[
            pl.BlockSpec((1, 1, 4), lambda h: (h, 0, 0)),
            pl.BlockSpec((N_DET, 4), lambda h: (0, 0)),
        ],
        out_specs=pl.BlockSpec((8 * N_DET, 2), lambda h: (h, 0)),
        out_shape=jax.ShapeDtypeStruct((P * 8, 2), jnp.int32),
    )(boxes[:N_HUM].reshape(N_HUM, 1, 4), boxes)


def _sc_gather(featT, idx_flat, n):
    mesh = plsc.VectorSubcoreMesh(core_axis_name="c", subcore_axis_name="s")

    per_w = n // SC_WORKERS
    n_ch = per_w // CW

    @functools.partial(
        pl.kernel,
        out_type=jax.ShapeDtypeStruct((n, C), jnp.float32),
        mesh=mesh,
        compiler_params=pltpu.CompilerParams(use_tc_tiling_on_sc=False),
        scratch_types=[
            pltpu.VMEM((1, per_w), jnp.int32),
            pltpu.VMEM((CW, C), jnp.float32),
            pltpu.VMEM((CW, C), jnp.float32),
            pltpu.SemaphoreType.DMA,
            pltpu.SemaphoreType.DMA,
            pltpu.SemaphoreType.DMA,
            pltpu.SemaphoreType.DMA,
        ],
    )
    def gk2(x_hbm, i_hbm, o_hbm, idx_v, buf0, buf1, gs0, gs1, os0, os1):
        cid = jax.lax.axis_index("c")
        sid = jax.lax.axis_index("s")
        wid = sid * 2 + cid
        base = wid * per_w
        pltpu.sync_copy(i_hbm.at[0, pl.ds(base, per_w)], idx_v.at[0])
        bufs, gsems, osems = (buf0, buf1), (gs0, gs1), (os0, os1)
        gathers, outs = [None] * n_ch, [None] * n_ch
        for k in range(n_ch + 2):
            if k < n_ch:
                if k >= 2:
                    outs[k - 2].wait()
                gathers[k] = pltpu.make_async_copy(
                    x_hbm.at[idx_v.at[0, pl.ds(k * CW, CW)]],
                    bufs[k % 2],
                    gsems[k % 2],
                )
                gathers[k].start()
            if 1 <= k <= n_ch:
                j = k - 1
                gathers[j].wait()
                outs[j] = pltpu.make_async_copy(
                    bufs[j % 2],
                    o_hbm.at[pl.ds(base + j * CW, CW)],
                    osems[j % 2],
                )
                outs[j].start()
        outs[n_ch - 2].wait()
        outs[n_ch - 1].wait()

    return gk2(featT, idx_flat)


def _head_body(h_base, x_ref, sc_ref, lab_ref, o2t_ref,
               w1_ref, b1_ref, w2_ref, b2_ref, w3_ref, b3_ref, out_ref):
    h = pl.program_id(0) + h_base
    rowi = jax.lax.broadcasted_iota(jnp.int32, (N_DET, 1), 0)
    hm = rowi == h
    s = sc_ref[...]
    se = jnp.where(s >= THRESH, s, 0.0)
    sh = jnp.sum(jnp.where(hm, se, 0.0))
    ds = sh * se * jnp.where(hm, 0.0, 1.0)
    lab = lab_ref[...]
    l_iota = jax.lax.broadcasted_iota(jnp.int32, (N_DET, NUM_OBJ), 1).astype(
        jnp.float32)
    ohl = jnp.where(lab == l_iota, 1.0, 0.0)
    tgt = jnp.sum(ohl * o2t_ref[...], axis=1, keepdims=True)

    x8 = x_ref[...]
    x = jnp.concatenate(
        [x8[g8 * N_DET:(g8 + 1) * N_DET, :] for g8 in range(NPTS // 2)],
        axis=1,
    )
    h1 = jax.nn.relu(
        jnp.dot(x, w1_ref[...], preferred_element_type=jnp.float32)
        + b1_ref[...]
    )
    h2 = jax.nn.relu(
        jnp.dot(h1, w2_ref[...], preferred_element_type=jnp.float32)
        + b2_ref[...]
    )
    logits = (
        jnp.dot(h2, w3_ref[...], preferred_element_type=jnp.float32)
        + b3_ref[...]
    )
    k_iota = jax.lax.broadcasted_iota(jnp.int32, (N_DET, NUM_CLASSES), 1
                                      ).astype(jnp.float32)
    onehot = jnp.where(tgt == k_iota, 1.0, 0.0)
    out_ref[...] = ds * onehot * jax.nn.sigmoid(logits)


def _head(h_base, n_h, x2d, scores_c, labels_f, o2t_f, W1p, b1r, W2, b2r,
          W3, b3r):
    full = lambda shape: pl.BlockSpec(shape, lambda h: (0, 0))
    return pl.pallas_call(
        functools.partial(_head_body, h_base),
        grid=(n_h,),
        in_specs=[
            pl.BlockSpec((8 * N_DET, 2 * C), lambda h: (h, 0)),
            full((N_DET, 1)),
            full((N_DET, 1)),
            full((1, NUM_OBJ)),
            full((C * NPTS, REP)),
            full((1, REP)),
            full((REP, REP)),
            full((1, REP)),
            full((REP, NUM_CLASSES)),
            full((1, NUM_CLASSES)),
        ],
        out_specs=pl.BlockSpec((N_DET, NUM_CLASSES), lambda h: (h, 0)),
        out_shape=jax.ShapeDtypeStruct((n_h * N_DET, NUM_CLASSES),
                                       jnp.float32),
    )(x2d, scores_c, labels_f, o2t_f, W1p, b1r, W2, b2r, W3, b3r)


def kernel(features, boxes, scores, labels, W1, b1, W2, b2, W3, b3, obj2target):
    featT = features.transpose(1, 2, 0).reshape(FH * FW, C)
    W1p = (W1.reshape(C, NPTS // 2, 2, REP).transpose(1, 2, 0, 3)
           .reshape(C * NPTS, REP))
    idx = _pair_indices(boxes)
    scores_c = scores.reshape(N_DET, 1)
    labels_f = labels.astype(jnp.float32).reshape(N_DET, 1)
    o2t_f = obj2target.astype(jnp.float32).reshape(1, NUM_OBJ)
    b1r, b2r, b3r = b1.reshape(1, REP), b2.reshape(1, REP), b3.reshape(1, NUM_CLASSES)
    h_per = N_HUM // CHUNKS
    rows_per = h_per * N_DET * 8
    n_per = NIDX // CHUNKS
    outs = []
    for k in range(CHUNKS):
        idx_k = idx[k * rows_per:(k + 1) * rows_per].reshape(1, n_per)
        pooled = _sc_gather(featT, idx_k, n_per)
        x2d = pooled.reshape(n_per // 2, 2 * C)
        outs.append(_head(k * h_per, h_per, x2d, scores_c, labels_f, o2t_f,
                          W1p, b1r, W2, b2r, W3, b3r))
    return jnp.concatenate(outs, axis=0)


# speedup vs baseline: 1.7463x; 1.4427x over previous
import functools

import numpy as np

import jax
import jax.numpy as jnp
from jax.experimental import pallas as pl
from jax.experimental.pallas import tpu as pltpu
from jax.experimental.pallas import tpu_sc as plsc

NUM_CLASSES = 117
NUM_OBJ = 80
N_DET = 1000
N_HUM = 16
POOL = 4
NPTS = POOL * POOL
C = 64
FH = FW = 64
THRESH = 0.2
P = N_HUM * N_DET
NIDX = P * NPTS
REP = 128
GATHER_WIN = 256
CHUNKS = 1
SC_WORKERS = 32
CW = 400


def _idx_body(bh_ref, x1o_ref, y1o_ref, x2o_ref, y2o_ref, fy_ref, fx_ref, out_ref):
    x1o, y1o = x1o_ref[...], y1o_ref[...]
    x2o, y2o = x2o_ref[...], y2o_ref[...]
    fy, fx = fy_ref[...], fx_ref[...]
    rows = []
    for hh in range(8):
        bh = bh_ref[hh]
        ux1 = jnp.minimum(bh[:, 0:1], x1o)
        uy1 = jnp.minimum(bh[:, 1:2], y1o)
        ux2 = jnp.maximum(bh[:, 2:3], x2o)
        uy2 = jnp.maximum(bh[:, 3:4], y2o)
        yi = jnp.clip(jnp.round(uy1 + (uy2 - uy1) * fy), 0.0, FH - 1)
        xi = jnp.clip(jnp.round(ux1 + (ux2 - ux1) * fx), 0.0, FW - 1)
        rows.append((yi * FW + xi).astype(jnp.int32))
    out_ref[...] = jnp.concatenate(rows, axis=0)


_ROWS_H = P * NPTS // N_HUM // 128
_q = np.arange(NPTS * N_DET)
_g8, _s = _q // (2 * N_DET), _q % 2
_FY = jnp.asarray(((_g8 // 2 + 0.5) / POOL).reshape(_ROWS_H, 128),
                  dtype=jnp.float32)
_FX = jnp.asarray(((2 * (_g8 % 2) + _s + 0.5) / POOL).reshape(_ROWS_H, 128),
                  dtype=jnp.float32)


def _pair_indices(boxes):
    ob = lambda col: jnp.tile(
        jnp.repeat(boxes[:, col], 2), NPTS // 2).reshape(_ROWS_H, 128)
    full = lambda: pl.BlockSpec((_ROWS_H, 128), lambda t: (0, 0))
    return pl.pallas_call(
        _idx_body,
        grid=(N_HUM // 8,),
        in_specs=[
            pl.BlockSpec((8, 1, 4), lambda t: (t, 0, 0)),
            full(), full(), full(), full(), full(), full(),
        ],
        out_specs=pl.BlockSpec((8 * _ROWS_H, 128), lambda t: (t, 0)),
        out_shape=jax.ShapeDtypeStruct((N_HUM * _ROWS_H, 128), jnp.int32),
    )(boxes[:N_HUM].reshape(N_HUM, 1, 4), ob(0), ob(1), ob(2), ob(3),
      _FY, _FX)


def _sc_gather(featT, idx_flat, n):
    mesh = plsc.VectorSubcoreMesh(core_axis_name="c", subcore_axis_name="s")

    per_w = n // SC_WORKERS
    n_ch = per_w // CW

    @functools.partial(
        pl.kernel,
        out_type=jax.ShapeDtypeStruct((n, C), jnp.float32),
        mesh=mesh,
        compiler_params=pltpu.CompilerParams(use_tc_tiling_on_sc=False),
        scratch_types=[
            pltpu.VMEM((1, per_w), jnp.int32),
            pltpu.VMEM((CW, C), jnp.float32),
            pltpu.VMEM((CW, C), jnp.float32),
            pltpu.SemaphoreType.DMA,
            pltpu.SemaphoreType.DMA,
            pltpu.SemaphoreType.DMA,
            pltpu.SemaphoreType.DMA,
        ],
    )
    def gk2(x_hbm, i_hbm, o_hbm, idx_v, buf0, buf1, gs0, gs1, os0, os1):
        cid = jax.lax.axis_index("c")
        sid = jax.lax.axis_index("s")
        wid = sid * 2 + cid
        base = wid * per_w
        pltpu.sync_copy(i_hbm.at[0, pl.ds(base, per_w)], idx_v.at[0])
        bufs, gsems, osems = (buf0, buf1), (gs0, gs1), (os0, os1)
        gathers, outs = [None] * n_ch, [None] * n_ch
        for k in range(n_ch + 2):
            if k < n_ch:
                if k >= 2:
                    outs[k - 2].wait()
                gathers[k] = pltpu.make_async_copy(
                    x_hbm.at[idx_v.at[0, pl.ds(k * CW, CW)]],
                    bufs[k % 2],
                    gsems[k % 2],
                )
                gathers[k].start()
            if 1 <= k <= n_ch:
                j = k - 1
                gathers[j].wait()
                outs[j] = pltpu.make_async_copy(
                    bufs[j % 2],
                    o_hbm.at[pl.ds(base + j * CW, CW)],
                    osems[j % 2],
                )
                outs[j].start()
        outs[n_ch - 2].wait()
        outs[n_ch - 1].wait()

    return gk2(featT, idx_flat)


def _head_body(h_base, x_ref, sc_ref, lab_ref, o2t_ref,
               w1_ref, b1_ref, w2_ref, b2_ref, w3_ref, b3_ref, out_ref):
    h = pl.program_id(0) + h_base
    rowi = jax.lax.broadcasted_iota(jnp.int32, (N_DET, 1), 0)
    hm = rowi == h
    s = sc_ref[...]
    se = jnp.where(s >= THRESH, s, 0.0)
    sh = jnp.sum(jnp.where(hm, se, 0.0))
    ds = sh * se * jnp.where(hm, 0.0, 1.0)
    lab = lab_ref[...]
    l_iota = jax.lax.broadcasted_iota(jnp.int32, (N_DET, NUM_OBJ), 1).astype(
        jnp.float32)
    ohl = jnp.where(lab == l_iota, 1.0, 0.0)
    tgt = jnp.sum(ohl * o2t_ref[...], axis=1, keepdims=True)

    x8 = x_ref[...]
    x = jnp.concatenate(
        [x8[g8 * N_DET:(g8 + 1) * N_DET, :] for g8 in range(NPTS // 2)],
        axis=1,
    )
    h1 = jax.nn.relu(
        jnp.dot(x, w1_ref[...], preferred_element_type=jnp.float32)
        + b1_ref[...]
    )
    h2 = jax.nn.relu(
        jnp.dot(h1, w2_ref[...], preferred_element_type=jnp.float32)
        + b2_ref[...]
    )
    logits = (
        jnp.dot(h2, w3_ref[...], preferred_element_type=jnp.float32)
        + b3_ref[...]
    )
    k_iota = jax.lax.broadcasted_iota(jnp.int32, (N_DET, NUM_CLASSES), 1
                                      ).astype(jnp.float32)
    onehot = jnp.where(tgt == k_iota, 1.0, 0.0)
    out_ref[...] = ds * onehot * jax.nn.sigmoid(logits)


def _head(h_base, n_h, x2d, scores_c, labels_f, o2t_f, W1p, b1r, W2, b2r,
          W3, b3r):
    full = lambda shape: pl.BlockSpec(shape, lambda h: (0, 0))
    return pl.pallas_call(
        functools.partial(_head_body, h_base),
        grid=(n_h,),
        in_specs=[
            pl.BlockSpec((8 * N_DET, 2 * C), lambda h: (h, 0)),
            full((N_DET, 1)),
            full((N_DET, 1)),
            full((1, NUM_OBJ)),
            full((C * NPTS, REP)),
            full((1, REP)),
            full((REP, REP)),
            full((1, REP)),
            full((REP, NUM_CLASSES)),
            full((1, NUM_CLASSES)),
        ],
        out_specs=pl.BlockSpec((N_DET, NUM_CLASSES), lambda h: (h, 0)),
        out_shape=jax.ShapeDtypeStruct((n_h * N_DET, NUM_CLASSES),
                                       jnp.float32),
    )(x2d, scores_c, labels_f, o2t_f, W1p, b1r, W2, b2r, W3, b3r)


def kernel(features, boxes, scores, labels, W1, b1, W2, b2, W3, b3, obj2target):
    featT = features.transpose(1, 2, 0).reshape(FH * FW, C)
    W1p = (W1.reshape(C, NPTS // 2, 2, REP).transpose(1, 2, 0, 3)
           .reshape(C * NPTS, REP))
    idx = _pair_indices(boxes)
    scores_c = scores.reshape(N_DET, 1)
    labels_f = labels.astype(jnp.float32).reshape(N_DET, 1)
    o2t_f = obj2target.astype(jnp.float32).reshape(1, NUM_OBJ)
    b1r, b2r, b3r = b1.reshape(1, REP), b2.reshape(1, REP), b3.reshape(1, NUM_CLASSES)
    h_per = N_HUM // CHUNKS
    rows_per = h_per * _ROWS_H
    n_per = NIDX // CHUNKS
    outs = []
    for k in range(CHUNKS):
        idx_k = idx[k * rows_per:(k + 1) * rows_per].reshape(1, n_per)
        pooled = _sc_gather(featT, idx_k, n_per)
        x2d = pooled.reshape(n_per // 2, 2 * C)
        outs.append(_head(k * h_per, h_per, x2d, scores_c, labels_f, o2t_f,
                          W1p, b1r, W2, b2r, W3, b3r))
    return jnp.concatenate(outs, axis=0)
